# Initial kernel scaffold; baseline (speedup 1.0000x reference)
import jax
import jax.numpy as jnp
from jax.experimental import pallas as pl


def kernel(x, embedding):
    raise NotImplementedError("write your pallas kernel here")


# speedup vs baseline: 1.8437x; 1.8437x over previous
import functools

import jax
import jax.numpy as jnp
from jax import lax
from jax.experimental import pallas as pl
from jax.experimental.pallas import tpu as pltpu
from jax.experimental.pallas import tpu_sc as plsc

_INFO = plsc.get_sparse_core_info()
_NC, _NS = _INFO.num_cores, _INFO.num_subcores
_NW = _NC * _NS


@functools.lru_cache(maxsize=None)
def _make_gather(B, D, C):
    b_per_w = B // _NW
    n_chunks = b_per_w // C
    mesh = plsc.VectorSubcoreMesh(core_axis_name="c", subcore_axis_name="s")

    @functools.partial(
        pl.kernel,
        mesh=mesh,
        out_type=jax.ShapeDtypeStruct((B, D), jnp.float32),
        scratch_types=[
            pltpu.VMEM((C,), jnp.int32),
            pltpu.VMEM((C, D), jnp.float32),
            pltpu.SemaphoreType.DMA,
        ],
        compiler_params=pltpu.CompilerParams(use_tc_tiling_on_sc=False),
    )
    def gather_kernel(idx_hbm, table_hbm, out_hbm, idx_v, rows_v, sem):
        wid = lax.axis_index("s") * _NC + lax.axis_index("c")
        base = wid * b_per_w

        def body(i, carry):
            off = base + i * C
            pltpu.sync_copy(idx_hbm.at[pl.ds(off, C)], idx_v)
            pltpu.async_copy(table_hbm.at[idx_v], rows_v, sem).wait()
            pltpu.sync_copy(rows_v, out_hbm.at[pl.ds(off, C)])
            return carry

        lax.fori_loop(0, n_chunks, body, 0)

    return gather_kernel


def kernel(x, embedding):
    batch, hist = x.shape
    dim = embedding.shape[1]
    flat_idx = x.reshape(batch * hist)
    out = _make_gather(batch * hist, dim, 1024)(flat_idx, embedding)
    return out.reshape(batch, hist, dim)


# speedup vs baseline: 1.8585x; 1.0080x over previous
import functools

import jax
import jax.numpy as jnp
from jax import lax
from jax.experimental import pallas as pl
from jax.experimental.pallas import tpu as pltpu
from jax.experimental.pallas import tpu_sc as plsc

_INFO = plsc.get_sparse_core_info()
_NC, _NS = _INFO.num_cores, _INFO.num_subcores
_NW = _NC * _NS

_NBUF = 2


@functools.lru_cache(maxsize=None)
def _make_gather(B, D, C):
    b_per_w = B // _NW
    n_chunks = b_per_w // C
    n_groups = n_chunks // _NBUF
    mesh = plsc.VectorSubcoreMesh(core_axis_name="c", subcore_axis_name="s")

    scratch = (
        [pltpu.VMEM((C,), jnp.int32) for _ in range(_NBUF)]
        + [pltpu.VMEM((C, D), jnp.float32) for _ in range(_NBUF)]
        + [pltpu.SemaphoreType.DMA for _ in range(3 * _NBUF)]
    )

    @functools.partial(
        pl.kernel,
        mesh=mesh,
        out_type=jax.ShapeDtypeStruct((B, D), jnp.float32),
        scratch_types=scratch,
        compiler_params=pltpu.CompilerParams(use_tc_tiling_on_sc=False),
    )
    def gather_kernel(idx_hbm, table_hbm, out_hbm, *bufs):
        idx_v = bufs[:_NBUF]
        rows_v = bufs[_NBUF : 2 * _NBUF]
        sem_i = bufs[2 * _NBUF : 3 * _NBUF]
        sem_g = bufs[3 * _NBUF : 4 * _NBUF]
        sem_o = bufs[4 * _NBUF : 5 * _NBUF]

        wid = lax.axis_index("s") * _NC + lax.axis_index("c")
        base = wid * b_per_w

        for b in range(_NBUF):
            pltpu.async_copy(idx_hbm.at[pl.ds(base + b * C, C)], idx_v[b], sem_i[b])

        def body(g, carry):
            for b in range(_NBUF):
                chunk = g * _NBUF + b
                off = base + chunk * C
                pltpu.make_async_copy(
                    idx_hbm.at[pl.ds(off, C)], idx_v[b], sem_i[b]
                ).wait()

                @pl.when(g > 0)
                def _wait_prev_writeback(b=b, off=off):
                    pltpu.make_async_copy(
                        rows_v[b], out_hbm.at[pl.ds(off - _NBUF * C, C)], sem_o[b]
                    ).wait()

                pltpu.async_copy(table_hbm.at[idx_v[b]], rows_v[b], sem_g[b])

            for b in range(_NBUF):
                chunk = g * _NBUF + b
                off = base + chunk * C
                pltpu.make_async_copy(
                    table_hbm.at[idx_v[b]], rows_v[b], sem_g[b]
                ).wait()

                @pl.when(chunk + _NBUF < n_chunks)
                def _prefetch_idx(b=b, off=off):
                    pltpu.async_copy(
                        idx_hbm.at[pl.ds(off + _NBUF * C, C)], idx_v[b], sem_i[b]
                    )

                pltpu.async_copy(rows_v[b], out_hbm.at[pl.ds(off, C)], sem_o[b])
            return carry

        lax.fori_loop(0, n_groups, body, 0)

        last = base + (n_chunks - _NBUF) * C
        for b in range(_NBUF):
            pltpu.make_async_copy(
                rows_v[b], out_hbm.at[pl.ds(last + b * C, C)], sem_o[b]
            ).wait()

    return gather_kernel


def kernel(x, embedding):
    batch, hist = x.shape
    dim = embedding.shape[1]
    flat_idx = x.reshape(batch * hist)
    out = _make_gather(batch * hist, dim, 800)(flat_idx, embedding)
    return out.reshape(batch, hist, dim)
